# baseline (speedup 1.0000x reference)
import functools

import jax
import jax.numpy as jnp
from jax import lax
from jax.experimental import pallas as pl
from jax.experimental.pallas import tpu as pltpu
from jax.experimental.pallas import tpu_sc as plsc

H, M = 16, 2048
BLK_R = 1024
N_BLK = M // BLK_R
NEG_INF = float("-inf")


NC, NS, LANES = 2, 16, 16
NW = NC * NS
PER_W = M // 2
TILES_W = PER_W // 8


N_BAND = PER_W // 128


def _sc_diag_body(x_hbm, out_hbm, band_v, acc_v, sem):
    wid = lax.axis_index("s") * NC + lax.axis_index("c")
    head = wid // 2
    half = wid % 2
    r0 = head * M + half * PER_W
    c0 = half * PER_W
    p16 = lax.iota(jnp.int32, LANES)

    def fire(bd, buf):
        return pltpu.async_copy(
            x_hbm.at[pl.ds(r0 + bd * 128, 128), pl.ds(c0 + bd * 128, 128)],
            band_v.at[buf],
            sem,
        )

    acc = jnp.full((LANES,), NEG_INF, jnp.float32)
    cur = fire(0, 0)
    for bd in range(N_BAND):
        nxt = fire(bd + 1, (bd + 1) % 2) if bd + 1 < N_BAND else None
        cur.wait()
        for j in range(128):
            v = band_v[bd % 2, j, pl.ds((j // 16) * 16, 16)]
            acc = jnp.maximum(acc, jnp.where(p16 == j % 16, v, NEG_INF))
        cur = nxt
    acc_v[...] = acc
    pltpu.sync_copy(acc_v, out_hbm.at[wid])


def _sc_diag(x2v):
    mesh = plsc.VectorSubcoreMesh(core_axis_name="c", subcore_axis_name="s")
    k = functools.partial(
        pl.kernel,
        mesh=mesh,
        out_type=jax.ShapeDtypeStruct((NW, LANES), jnp.float32),
        scratch_types=[
            pltpu.VMEM((2, 128, 128), jnp.float32),
            pltpu.VMEM((LANES,), jnp.float32),
            pltpu.SemaphoreType.DMA,
        ],
        compiler_params=pltpu.CompilerParams(use_tc_tiling_on_sc=True),
    )(_sc_diag_body)
    return k(x2v)


def _tc_body(x_ref, off_ref):
    b = pl.program_id(1)
    blk = x_ref[0]
    stripe = x_ref[0, :, pl.ds(b * BLK_R, BLK_R)]
    eye = (
        lax.broadcasted_iota(jnp.int32, (BLK_R, BLK_R), 0)
        == lax.broadcasted_iota(jnp.int32, (BLK_R, BLK_R), 1)
    )
    stripe_off = jnp.max(jnp.where(eye, NEG_INF, stripe))
    colmax = jnp.max(blk, axis=0, keepdims=True)
    in_stripe = (lax.broadcasted_iota(jnp.int32, (1, M), 1) // BLK_R) == b
    off_m = jnp.maximum(jnp.max(jnp.where(in_stripe, NEG_INF, colmax)), stripe_off)

    @pl.when(b == 0)
    def _():
        off_ref[...] = jnp.full((1, 1, 128), NEG_INF, jnp.float32)

    off_ref[...] = jnp.maximum(off_ref[...], off_m)


def kernel(x):
    xs = x.reshape(H, M, M)
    diag_parts = _sc_diag(x.reshape(H * M, M))
    off = pl.pallas_call(
        _tc_body,
        grid=(H, N_BLK),
        in_specs=[pl.BlockSpec((1, BLK_R, M), lambda h, b: (h, b, 0))],
        out_specs=[pl.BlockSpec((1, 1, 128), lambda h, b: (h, 0, 0))],
        out_shape=[jax.ShapeDtypeStruct((H, 1, 128), jnp.float32)],
    )(xs)[0]
    max_diag = jnp.max(diag_parts.reshape(H, 2 * LANES), axis=-1)
    return jnp.concatenate([max_diag, off[:, 0, 0]])[None, :]


# speedup vs baseline: 1.0093x; 1.0093x over previous
import functools

import jax
import jax.numpy as jnp
from jax import lax
from jax.experimental import pallas as pl
from jax.experimental.pallas import tpu as pltpu
from jax.experimental.pallas import tpu_sc as plsc

H, M = 16, 2048
BLK_R = 1024
N_BLK = M // BLK_R
NEG_INF = float("-inf")


NC, NS, LANES = 2, 16, 16
NW = NC * NS
PER_W = M // 2


N_BAND = PER_W // 128


def _sc_diag_body(x_hbm, out_hbm, band_v, acc_v, sem):
    wid = lax.axis_index("s") * NC + lax.axis_index("c")
    head = wid // 2
    half = wid % 2
    r0 = head * M + half * PER_W
    c0 = half * PER_W
    p16 = lax.iota(jnp.int32, LANES)

    def band_step(bd, acc):
        pltpu.async_copy(
            x_hbm.at[pl.ds(r0 + bd * 128, 128), pl.ds(c0 + bd * 128, 128)],
            band_v,
            sem,
        ).wait()
        for j in range(128):
            v = band_v[j, pl.ds((j // 16) * 16, 16)]
            acc = jnp.maximum(acc, jnp.where(p16 == j % 16, v, NEG_INF))
        return acc

    acc = jnp.full((LANES,), NEG_INF, jnp.float32)
    acc = lax.fori_loop(0, N_BAND, band_step, acc)
    acc_v[...] = acc
    pltpu.sync_copy(acc_v, out_hbm.at[wid])


def _sc_diag(x2v):
    mesh = plsc.VectorSubcoreMesh(core_axis_name="c", subcore_axis_name="s")
    k = functools.partial(
        pl.kernel,
        mesh=mesh,
        out_type=jax.ShapeDtypeStruct((NW, LANES), jnp.float32),
        scratch_types=[
            pltpu.VMEM((128, 128), jnp.float32),
            pltpu.VMEM((LANES,), jnp.float32),
            pltpu.SemaphoreType.DMA,
        ],
        compiler_params=pltpu.CompilerParams(use_tc_tiling_on_sc=True),
    )(_sc_diag_body)
    return k(x2v)


def _tc_body(x_ref, off_ref):
    b = pl.program_id(1)
    blk = x_ref[0]
    stripe = x_ref[0, :, pl.ds(b * BLK_R, BLK_R)]
    eye = (
        lax.broadcasted_iota(jnp.int32, (BLK_R, BLK_R), 0)
        == lax.broadcasted_iota(jnp.int32, (BLK_R, BLK_R), 1)
    )
    stripe_off = jnp.max(jnp.where(eye, NEG_INF, stripe))
    colmax = jnp.max(blk, axis=0, keepdims=True)
    in_stripe = (lax.broadcasted_iota(jnp.int32, (1, M), 1) // BLK_R) == b
    off_m = jnp.maximum(jnp.max(jnp.where(in_stripe, NEG_INF, colmax)), stripe_off)

    @pl.when(b == 0)
    def _():
        off_ref[...] = jnp.full((1, 1, 128), NEG_INF, jnp.float32)

    off_ref[...] = jnp.maximum(off_ref[...], off_m)


def kernel(x):
    xs = x.reshape(H, M, M)
    diag_parts = _sc_diag(x.reshape(H * M, M))
    off = pl.pallas_call(
        _tc_body,
        grid=(H, N_BLK),
        in_specs=[pl.BlockSpec((1, BLK_R, M), lambda h, b: (h, b, 0))],
        out_specs=[pl.BlockSpec((1, 1, 128), lambda h, b: (h, 0, 0))],
        out_shape=[jax.ShapeDtypeStruct((H, 1, 128), jnp.float32)],
    )(xs)[0]
    max_diag = jnp.max(diag_parts.reshape(H, 2 * LANES), axis=-1)
    return jnp.concatenate([max_diag, off[:, 0, 0]])[None, :]
